# baseline (speedup 1.0000x reference)
import functools

import jax
import jax.numpy as jnp
from jax import lax
from jax.experimental import pallas as pl
from jax.experimental.pallas import tpu as pltpu
from jax.experimental.pallas import tpu_sc as plsc

VOCAB = 1000000
EMBED = 32
BATCH = 16384
HIST = 50

NC = 2
NS = 16
NW = NC * NS

LINES = BATCH // 2
L_PER_W = LINES // NW
GKP = 8
N_CHUNK = L_PER_W // GKP

GROUPS = (0, 16, 32, 40, 56, 72, 88, 96)


def _make_kernel():
  mesh = plsc.VectorSubcoreMesh(core_axis_name="c", subcore_axis_name="s")

  @functools.partial(
      pl.kernel,
      out_type=jax.ShapeDtypeStruct((BATCH, HIST, EMBED), jnp.float32),
      mesh=mesh,
      compiler_params=pltpu.CompilerParams(needs_layout_passes=False),
      scratch_types=[
          pltpu.VMEM((GKP, 128), jnp.int32),
          pltpu.VMEM((GKP, 128), jnp.int32),
          pltpu.VMEM((GKP, 128), jnp.int32),
          pltpu.VMEM((2, 128, 128), jnp.float32),
          pltpu.VMEM((128, EMBED), jnp.float32),
          pltpu.SemaphoreType.DMA,
          pltpu.SemaphoreType.DMA,
      ],
  )
  def gather_kernel(x2_hbm, t2_hbm, out_hbm, xv, qv, cv, dst, outv, sem0, sem1):
    sems = (sem0, sem1)
    wid = lax.axis_index("s") * NC + lax.axis_index("c")
    base = wid * L_PER_W
    iota = lax.iota(jnp.int32, 16)

    @pl.loop(0, N_CHUNK)
    def _chunk(ch):
      r0 = base + ch * GKP
      pltpu.sync_copy(x2_hbm.at[pl.ds(r0, GKP)], xv)
      for p in range(GKP):
        for g in range(8):
          v = xv[p, pl.ds(g * 16, 16)]
          qv[p, pl.ds(g * 16, 16)] = lax.shift_right_logical(v, 2)
          cv[p, pl.ds(g * 16, 16)] = lax.shift_left(jnp.bitwise_and(v, 3), 5)
      pltpu.async_copy(t2_hbm.at[qv.at[0]], dst.at[0], sems[0])
      for p in range(GKP):
        if p + 1 < GKP:
          pltpu.async_copy(
              t2_hbm.at[qv.at[p + 1]], dst.at[(p + 1) % 2], sems[(p + 1) % 2])
        pltpu.make_async_copy(
            t2_hbm.at[qv.at[p]], dst.at[p % 2], sems[p % 2]).wait()
        dstp = dst.at[p % 2]
        for j0 in GROUPS:
          rows = j0 + iota
          cvec = cv[p, pl.ds(j0, 16)]
          for e in range(EMBED):
            evec = jnp.full((16,), e, jnp.int32)
            val = plsc.load_gather(dstp, [rows, cvec + evec])
            plsc.store_scatter(outv, [rows, evec], val)
        b = 2 * (r0 + p)
        pltpu.sync_copy(outv.at[pl.ds(0, HIST)], out_hbm.at[b])
        pltpu.sync_copy(outv.at[pl.ds(56, HIST)], out_hbm.at[b + 1])

  return gather_kernel


_gather = _make_kernel()


@jax.jit
def kernel(x, table):
  t2 = table.reshape(VOCAB // 4, 128)
  xr = x.reshape(LINES, 2, HIST)
  x2 = jnp.pad(xr, ((0, 0), (0, 0), (0, 6))).reshape(LINES, 112)
  x2 = jnp.pad(x2, ((0, 0), (0, 16)))
  return _gather(x2, t2)


# speedup vs baseline: 7.7714x; 7.7714x over previous
import functools

import jax
import jax.numpy as jnp
from jax import lax
from jax.experimental import pallas as pl
from jax.experimental.pallas import tpu as pltpu
from jax.experimental.pallas import tpu_sc as plsc

VOCAB = 1000000
EMBED = 32
BATCH = 16384
HIST = 50

NC = 2
NS = 16
NW = NC * NS

BB = 128
NBLK = BATCH // BB
BLK_PER_W = NBLK // NW
DSTW = 128


def _make_kernel():
  mesh = plsc.VectorSubcoreMesh(core_axis_name="c", subcore_axis_name="s")

  @functools.partial(
      pl.kernel,
      out_type=jax.ShapeDtypeStruct((HIST, EMBED, BATCH), jnp.float32),
      mesh=mesh,
      compiler_params=pltpu.CompilerParams(needs_layout_passes=False),
      scratch_types=[
          pltpu.VMEM((56, BB), jnp.int32),
          pltpu.VMEM((BB,), jnp.int32),
          pltpu.VMEM((BB,), jnp.int32),
          pltpu.VMEM((56, BB), jnp.int32),
          pltpu.VMEM((BB, DSTW), jnp.float32),
          pltpu.VMEM((BB, DSTW), jnp.float32),
          pltpu.VMEM((EMBED, BB), jnp.float32),
          pltpu.VMEM((EMBED, BB), jnp.float32),
          pltpu.SemaphoreType.DMA,
          pltpu.SemaphoreType.DMA,
          pltpu.SemaphoreType.DMA,
          pltpu.SemaphoreType.DMA,
      ],
  )
  def gather_kernel(xt_hbm, t2_hbm, out_hbm, xv, qv0, qv1, cv,
                    dst0, dst1, panel0, panel1, sg0, sg1, sp0, sp1):
    qvs = (qv0, qv1)
    dsts = (dst0, dst1)
    panels = (panel0, panel1)
    sgs = (sg0, sg1)
    sps = (sp0, sp1)
    wid = lax.axis_index("s") * NC + lax.axis_index("c")
    iota = lax.iota(jnp.int32, 16)

    @pl.loop(0, BLK_PER_W)
    def _blk(blk):
      b0 = (wid * BLK_PER_W + blk) * BB
      for h0 in range(0, 48, 8):
        pltpu.sync_copy(xt_hbm.at[pl.ds(h0, 8), pl.ds(b0, BB)],
                        xv.at[pl.ds(h0, 8)])
      pltpu.sync_copy(xt_hbm.at[pl.ds(48, 2), pl.ds(b0, BB)],
                      xv.at[pl.ds(48, 2)])

      def line(h, slot):
        qv = qvs[slot]
        for g in range(8):
          v = xv[h, pl.ds(g * 16, 16)]
          qv[pl.ds(g * 16, 16)] = lax.shift_right_logical(v, 2)
          cv[h, pl.ds(g * 16, 16)] = lax.shift_left(jnp.bitwise_and(v, 3), 5)
        pltpu.async_copy(t2_hbm.at[qvs[slot]], dsts[slot], sgs[slot])

      def select(h, slot):
        pltpu.make_async_copy(t2_hbm.at[qvs[slot]], dsts[slot],
                              sgs[slot]).wait()
        dstp = dsts[slot]
        pan = panels[slot]
        for g in range(8):
          rows = g * 16 + iota
          cvec = cv[h, pl.ds(g * 16, 16)]
          for e in range(EMBED):
            val = plsc.load_gather(dstp, [rows, cvec + e])
            pan[e, pl.ds(g * 16, 16)] = val
        pltpu.async_copy(pan, out_hbm.at[h, :, pl.ds(b0, BB)], sps[slot])

      @pl.loop(0, HIST // 2)
      def _hpair(i):
        h0 = 2 * i
        h1 = 2 * i + 1
        line(h0, 0)
        line(h1, 1)
        select(h0, 0)
        select(h1, 1)
        pltpu.make_async_copy(
            panels[0], out_hbm.at[h0, :, pl.ds(b0, BB)], sps[0]).wait()
        pltpu.make_async_copy(
            panels[1], out_hbm.at[h1, :, pl.ds(b0, BB)], sps[1]).wait()

  return gather_kernel


_gather = _make_kernel()


@jax.jit
def kernel(x, table):
  xt = jnp.transpose(x)
  t2 = (jnp.transpose(table)
        .reshape(EMBED, VOCAB // 4, 4)
        .transpose(1, 2, 0)
        .reshape(VOCAB // 4, 128))
  out_t = _gather(xt, t2)
  return jnp.transpose(out_t, (2, 0, 1))
